# Initial kernel scaffold; baseline (speedup 1.0000x reference)
import jax
import jax.numpy as jnp
from jax.experimental import pallas as pl


def kernel(feature, img, edge_index, W_feat, b_feat, W_img, b_img, W_g1, b_g1, W_g2, b_g2):
    raise NotImplementedError("write your pallas kernel here")


# speedup vs baseline: 22.9296x; 22.9296x over previous
import functools

import jax
import jax.numpy as jnp
from jax import lax
from jax.experimental import pallas as pl
from jax.experimental.pallas import tpu as pltpu
from jax.experimental.pallas import tpu_sc as plsc

_N = 100000
_E = 1600000

_L = 128
_K = 8
_CHUNKS = 98
_ROWS_PER_TILE = _K * _CHUNKS
_E_PAD = _ROWS_PER_TILE * _L * 16

_KD = 8
_DEG_ROWS_PER_TILE = _KD * 49
_DEG_CHUNKS = 49

_ZCH = 6272
_NUP = _ZCH * 16
_ACC_ROWS = _NUP

_BLK = 2000
_GRID = _N // _BLK

_mesh = plsc.VectorSubcoreMesh(core_axis_name="c", subcore_axis_name="s",
                               num_cores=2, num_subcores=16)


def _writeback(acc, out, s):
    pltpu.sync_copy(acc.at[pl.ds(s * _ZCH, _ZCH)],
                    out.at[pl.ds(s * _ZCH, _ZCH)])


def _deg_body(dst2d, ones_h, z1d, out, didx, ones_v, acc):
    c = lax.axis_index("c")
    s = lax.axis_index("s")
    pltpu.sync_copy(z1d, acc.at[pl.ds(s * _ZCH, _ZCH)])
    pltpu.sync_copy(ones_h, ones_v)
    plsc.subcore_barrier()

    def body(i, carry):
        r0 = c * (_DEG_ROWS_PER_TILE * 16) + s * _DEG_ROWS_PER_TILE + i * _KD
        pltpu.sync_copy(dst2d.at[pl.ds(r0, _KD)], didx)
        for j in range(_KD):
            pltpu.sync_copy(ones_v, acc.at[didx.at[j]], add=True)
        return carry

    lax.fori_loop(0, _DEG_CHUNKS, body, 0)
    plsc.subcore_barrier()
    _writeback(acc, out.at[c], s)


@functools.partial(
    pl.kernel,
    out_type=jax.ShapeDtypeStruct((2, _NUP), jnp.float32),
    mesh=_mesh,
    scratch_types=[
        pltpu.VMEM((_KD, _L), jnp.int32),
        pltpu.VMEM((_L,), jnp.float32),
        pltpu.VMEM_SHARED((_ACC_ROWS,), jnp.float32),
    ],
)
def _deg(dst2d, ones_h, z1d, out, didx, ones_v, acc):
    _deg_body(dst2d, ones_h, z1d, out, didx, ones_v, acc)


def _scatter_body(src2d, dst2d, tlo, thi, zrows, out, sidx, didx, rows, acc,
                  sem):
    c = lax.axis_index("c")
    s = lax.axis_index("s")
    pltpu.sync_copy(zrows, acc.at[pl.ds(s * _ZCH, _ZCH)])
    plsc.subcore_barrier()

    def chunk(i, carry, table):
        r0 = s * _ROWS_PER_TILE + i * _K
        pltpu.sync_copy(src2d.at[pl.ds(r0, _K)], sidx)
        pltpu.sync_copy(dst2d.at[pl.ds(r0, _K)], didx)
        cps = [pltpu.async_copy(table.at[sidx.at[j]],
                                rows.at[pl.ds(j * _L, _L)], sem)
               for j in range(_K)]
        for cp in cps:
            cp.wait()
        for j in range(_K):
            pltpu.sync_copy(rows.at[pl.ds(j * _L, _L)], acc.at[didx.at[j]],
                            add=True)
        return carry

    @pl.when(c == 0)
    def _():
        lax.fori_loop(0, _CHUNKS, functools.partial(chunk, table=tlo), 0)

    @pl.when(c == 1)
    def _():
        lax.fori_loop(0, _CHUNKS, functools.partial(chunk, table=thi), 0)

    plsc.subcore_barrier()
    _writeback(acc, out.at[c], s)


@functools.partial(
    pl.kernel,
    out_type=jax.ShapeDtypeStruct((2, _NUP, 16), jnp.float32),
    mesh=_mesh,
    compiler_params=pltpu.CompilerParams(use_tc_tiling_on_sc=False),
    scratch_types=[
        pltpu.VMEM((_K, _L), jnp.int32),
        pltpu.VMEM((_K, _L), jnp.int32),
        pltpu.VMEM((_K * _L, 16), jnp.float32),
        pltpu.VMEM_SHARED((_ACC_ROWS, 16), jnp.float32),
        pltpu.SemaphoreType.DMA,
    ],
)
def _scatter(src2d, dst2d, tlo, thi, zrows, out, sidx, didx, rows, acc, sem):
    _scatter_body(src2d, dst2d, tlo, thi, zrows, out, sidx, didx, rows, acc,
                  sem)


def _dot(a, b):
    return jnp.dot(a, b, preferred_element_type=jnp.float32)


def _dinv_of(degT):
    deg = degT[:, 0:1] + degT[:, 1:2] + 1.0
    return lax.rsqrt(deg)


def _dense1_body(feat, img, degT, wf, bf, wi, bi, w1fl, w1fh, w1il, w1ih, out):
    dinv = _dinv_of(degT[...])
    f = jnp.maximum(_dot(feat[...], wf[...]) + bf[...], 0.0)
    im = jnp.maximum(_dot(img[...], wi[...]) + bi[...], 0.0)
    out[0] = (_dot(f, w1fl[...]) + _dot(im, w1il[...])) * dinv
    out[1] = (_dot(f, w1fh[...]) + _dot(im, w1ih[...])) * dinv


def _dense2_body(agg, ys, degT, w2ll, w2lh, w2hl, w2hh, b1l, b1h, out):
    dinv = _dinv_of(degT[...])
    hl = jnp.maximum((agg[0] + ys[0]) * dinv + b1l[...], 0.0)
    hh = jnp.maximum((agg[1] + ys[1]) * dinv + b1h[...], 0.0)
    out[0] = (_dot(hl, w2ll[...]) + _dot(hh, w2hl[...])) * dinv
    out[1] = (_dot(hl, w2lh[...]) + _dot(hh, w2hh[...])) * dinv


def _dense3_body(agg, ys, degT, b2l, b2h, out):
    dinv = _dinv_of(degT[...])
    out[0] = (agg[0] + ys[0]) * dinv + b2l[...]
    out[1] = (agg[1] + ys[1]) * dinv + b2h[...]


def _row_spec(cols):
    return pl.BlockSpec((_BLK, cols), lambda i: (i, 0))


def _full_spec(shape):
    nd = len(shape)
    return pl.BlockSpec(shape, lambda i, _n=nd: (0,) * _n)


def _half_spec():
    return pl.BlockSpec((2, _BLK, 16), lambda i: (0, i, 0))


def _dense1(feat, img, degT, wf, bf, wi, bi, w1fl, w1fh, w1il, w1ih):
    return pl.pallas_call(
        _dense1_body,
        grid=(_GRID,),
        in_specs=[_row_spec(32), _row_spec(32), _row_spec(2),
                  _full_spec((32, 24)), _full_spec((1, 24)),
                  _full_spec((32, 8)), _full_spec((1, 8)),
                  _full_spec((24, 16)), _full_spec((24, 16)),
                  _full_spec((8, 16)), _full_spec((8, 16))],
        out_specs=_half_spec(),
        out_shape=jax.ShapeDtypeStruct((2, _N, 16), jnp.float32),
    )(feat, img, degT, wf, bf, wi, bi, w1fl, w1fh, w1il, w1ih)


def _dense2(agg, ys, degT, w2ll, w2lh, w2hl, w2hh, b1l, b1h):
    return pl.pallas_call(
        _dense2_body,
        grid=(_GRID,),
        in_specs=[_half_spec(), _half_spec(), _row_spec(2),
                  _full_spec((16, 16)), _full_spec((16, 16)),
                  _full_spec((16, 16)), _full_spec((16, 16)),
                  _full_spec((1, 16)), _full_spec((1, 16))],
        out_specs=_half_spec(),
        out_shape=jax.ShapeDtypeStruct((2, _N, 16), jnp.float32),
    )(agg, ys, degT, w2ll, w2lh, w2hl, w2hh, b1l, b1h)


def _dense3(agg, ys, degT, b2l, b2h):
    return pl.pallas_call(
        _dense3_body,
        grid=(_GRID,),
        in_specs=[_half_spec(), _half_spec(), _row_spec(2),
                  _full_spec((1, 16)), _full_spec((1, 16))],
        out_specs=_half_spec(),
        out_shape=jax.ShapeDtypeStruct((2, _N, 16), jnp.float32),
    )(agg, ys, degT, b2l, b2h)


def kernel(feature, img, edge_index, W_feat, b_feat, W_img, b_img, W_g1, b_g1,
           W_g2, b_g2):
    pad = _E_PAD - _E
    src = jnp.concatenate([edge_index[0], jnp.zeros((pad,), jnp.int32)])
    dst = jnp.concatenate([edge_index[1], jnp.full((pad,), _N, jnp.int32)])
    src2d = src.reshape(-1, _L)
    dst2d = dst.reshape(-1, _L)

    ones_h = jnp.ones((_L,), jnp.float32)
    z1d = jnp.zeros((_ZCH,), jnp.float32)
    zrows = jnp.zeros((_ZCH, 16), jnp.float32)

    deg2 = _deg(dst2d, ones_h, z1d)
    degT = deg2[:, :_N].T

    w1fl, w1fh = W_g1[:24, :16], W_g1[:24, 16:]
    w1il, w1ih = W_g1[24:, :16], W_g1[24:, 16:]
    w2p = jnp.pad(W_g2, ((0, 0), (0, 12)))
    w2ll, w2lh = w2p[:16, :16], w2p[:16, 16:]
    w2hl, w2hh = w2p[16:, :16], w2p[16:, 16:]
    b1l, b1h = b_g1[:16].reshape(1, 16), b_g1[16:].reshape(1, 16)
    b2l = b_g2[:16].reshape(1, 16)
    b2h = jnp.pad(b_g2[16:], (0, 12)).reshape(1, 16)

    ys1 = _dense1(feature, img, degT, W_feat, b_feat.reshape(1, 24),
                  W_img, b_img.reshape(1, 8), w1fl, w1fh, w1il, w1ih)
    agg1 = _scatter(src2d, dst2d, ys1[0], ys1[1], zrows)
    ys2 = _dense2(agg1, ys1, degT, w2ll, w2lh, w2hl, w2hh, b1l, b1h)
    agg2 = _scatter(src2d, dst2d, ys2[0], ys2[1], zrows)
    out3 = _dense3(agg2, ys2, degT, b2l, b2h)
    return jnp.concatenate([out3[0], out3[1][:, :4]], axis=1)


# speedup vs baseline: 29.6142x; 1.2915x over previous
import functools

import jax
import jax.numpy as jnp
from jax import lax
from jax.experimental import pallas as pl
from jax.experimental.pallas import tpu as pltpu
from jax.experimental.pallas import tpu_sc as plsc

_N = 100000
_E = 1600000

_L = 128
_EROWS = _E // _L
_PADR = 44
_RPT = 784
_SLAB = 8
_SLABS = _RPT // _SLAB
_H = _SLAB // 2

_DRPW = 392
_DCH = 49

_ZCH = 6272
_NUP = _ZCH * 16

_BLK = 2000
_GRID = _N // _BLK

_mesh = plsc.VectorSubcoreMesh(core_axis_name="c", subcore_axis_name="s",
                               num_cores=2, num_subcores=16)


def _writeback(acc, out, s):
    pltpu.sync_copy(acc.at[pl.ds(s * _ZCH, _ZCH)],
                    out.at[pl.ds(s * _ZCH, _ZCH)])


def _deg_body(dst2d, ones_h, z1d, out, didx, ones_v, acc):
    c = lax.axis_index("c")
    s = lax.axis_index("s")
    w = c * 16 + s
    pltpu.sync_copy(z1d, acc.at[pl.ds(s * _ZCH, _ZCH)])
    pltpu.sync_copy(ones_h, ones_v)
    plsc.subcore_barrier()

    def body(i, carry):
        r0 = w * _DRPW + i * _SLAB
        pltpu.sync_copy(dst2d.at[pl.ds(r0, _SLAB)], didx)
        for j in range(_SLAB):
            pltpu.sync_copy(ones_v, acc.at[didx.at[j]], add=True)
        return carry

    lax.fori_loop(0, _DCH, body, 0)
    plsc.subcore_barrier()
    _writeback(acc, out.at[c], s)


@functools.partial(
    pl.kernel,
    out_type=jax.ShapeDtypeStruct((2, _NUP), jnp.float32),
    mesh=_mesh,
    scratch_types=[
        pltpu.VMEM((_SLAB, _L), jnp.int32),
        pltpu.VMEM((_L,), jnp.float32),
        pltpu.VMEM_SHARED((_NUP,), jnp.float32),
    ],
)
def _deg(dst2d, ones_h, z1d, out, didx, ones_v, acc):
    _deg_body(dst2d, ones_h, z1d, out, didx, ones_v, acc)


def _scatter_body(src2d, dst2d, tlo, thi, zrows, out, sidxA, didxA, sidxB,
                  didxB, rows0, rows1, acc, gsem0, gsem1, ssem0, ssem1, isem):
    c = lax.axis_index("c")
    s = lax.axis_index("s")
    pltpu.sync_copy(zrows, acc.at[pl.ds(s * _ZCH, _ZCH)])
    plsc.subcore_barrier()
    base = s * _RPT

    def idx_fetch(i, sidx, didx):
        r0 = base + i * _SLAB
        pltpu.async_copy(src2d.at[pl.ds(r0, _SLAB)], sidx, isem)
        pltpu.async_copy(dst2d.at[pl.ds(r0, _SLAB)], didx, isem)

    def idx_wait(sidx, didx):
        pltpu.make_async_copy(src2d.at[pl.ds(0, _SLAB)], sidx, isem).wait()
        pltpu.make_async_copy(dst2d.at[pl.ds(0, _SLAB)], didx, isem).wait()

    def run(table):
        def drain_g(rows, gsem):
            pltpu.make_async_copy(table.at[pl.ds(0, _H * _L)], rows,
                                  gsem).wait()

        def drain_s(rows, ssem):
            pltpu.make_async_copy(rows, acc.at[pl.ds(0, _H * _L)],
                                  ssem).wait()

        def do_slab(i, sidx, didx, sidxn, didxn, first):
            idx_wait(sidx, didx)

            def gathers(h, rows, gsem):
                for j in range(_H):
                    pltpu.async_copy(table.at[sidx.at[h * _H + j]],
                                     rows.at[pl.ds(j * _L, _L)], gsem)

            def scatters(h, rows, ssem):
                for j in range(_H):
                    pltpu.async_copy(rows.at[pl.ds(j * _L, _L)],
                                     acc.at[didx.at[h * _H + j]], ssem,
                                     add=True)

            if first is None:
                drain_s(rows0, ssem0)
            else:
                @pl.when(first)
                def _():
                    drain_s(rows0, ssem0)
            gathers(0, rows0, gsem0)
            if first is None:
                drain_s(rows1, ssem1)
            else:
                @pl.when(first)
                def _():
                    drain_s(rows1, ssem1)
            gathers(1, rows1, gsem1)
            idx_fetch(lax.min(i + 1, _SLABS - 1), sidxn, didxn)
            drain_g(rows0, gsem0)
            scatters(0, rows0, ssem0)
            drain_g(rows1, gsem1)
            scatters(1, rows1, ssem1)

        def body(k, carry):
            do_slab(2 * k, sidxA, didxA, sidxB, didxB, k > 0)
            do_slab(2 * k + 1, sidxB, didxB, sidxA, didxA, None)
            return carry

        idx_fetch(0, sidxA, didxA)
        lax.fori_loop(0, _SLABS // 2, body, 0)
        idx_wait(sidxA, didxA)
        drain_s(rows0, ssem0)
        drain_s(rows1, ssem1)

    @pl.when(c == 0)
    def _():
        run(tlo)

    @pl.when(c == 1)
    def _():
        run(thi)

    plsc.subcore_barrier()
    _writeback(acc, out.at[c], s)


@functools.partial(
    pl.kernel,
    out_type=jax.ShapeDtypeStruct((2, _NUP, 16), jnp.float32),
    mesh=_mesh,
    compiler_params=pltpu.CompilerParams(use_tc_tiling_on_sc=False),
    scratch_types=[
        pltpu.VMEM((_SLAB, _L), jnp.int32),
        pltpu.VMEM((_SLAB, _L), jnp.int32),
        pltpu.VMEM((_SLAB, _L), jnp.int32),
        pltpu.VMEM((_SLAB, _L), jnp.int32),
        pltpu.VMEM((_H * _L, 16), jnp.float32),
        pltpu.VMEM((_H * _L, 16), jnp.float32),
        pltpu.VMEM_SHARED((_NUP, 16), jnp.float32),
        pltpu.SemaphoreType.DMA,
        pltpu.SemaphoreType.DMA,
        pltpu.SemaphoreType.DMA,
        pltpu.SemaphoreType.DMA,
        pltpu.SemaphoreType.DMA,
    ],
)
def _scatter(src2d, dst2d, tlo, thi, zrows, out, sidxA, didxA, sidxB, didxB,
             rows0, rows1, acc, gsem0, gsem1, ssem0, ssem1, isem):
    _scatter_body(src2d, dst2d, tlo, thi, zrows, out, sidxA, didxA, sidxB,
                  didxB, rows0, rows1, acc, gsem0, gsem1, ssem0, ssem1, isem)


def _dot(a, b):
    return jnp.dot(a, b, preferred_element_type=jnp.float32)


def _dinv_of(degr):
    return lax.rsqrt(degr[0] + degr[1] + 1.0)


def _densea_body(feat, img, wf, wi, w1, ball, out):
    b = ball[...]
    f = jnp.maximum(_dot(feat[...], wf[...]) + b[0:1, 0:24], 0.0)
    im = jnp.maximum(_dot(img[...], wi[...]) + b[1:2, 0:8], 0.0)
    w1v = w1[...]
    out[0] = _dot(f, w1v[0:24, 0:16]) + _dot(im, w1v[24:32, 0:16])
    out[1] = _dot(f, w1v[0:24, 16:32]) + _dot(im, w1v[24:32, 16:32])


def _denseb_body(y1, degr, tlo, thi):
    dinv = _dinv_of(degr)
    tlo[...] = y1[0] * dinv
    thi[...] = y1[1] * dinv


def _dense2_body(agg, t1l, t1h, degr, w2, ball, tlo, thi):
    dinv = _dinv_of(degr)
    b = ball[...]
    hl = jnp.maximum((agg[0] + t1l[...]) * dinv + b[2:3, 0:16], 0.0)
    hh = jnp.maximum((agg[1] + t1h[...]) * dinv + b[2:3, 16:32], 0.0)
    w2v = w2[...]
    tlo[...] = (_dot(hl, w2v[0:16, 0:16]) + _dot(hh, w2v[16:32, 0:16])) * dinv
    thi[...] = (_dot(hl, w2v[0:16, 16:32]) + _dot(hh, w2v[16:32, 16:32])) * dinv


def _dense3_body(agg, t2l, t2h, degr, ball, olo, ohi):
    dinv = _dinv_of(degr)
    b = ball[...]
    olo[...] = (agg[0] + t2l[...]) * dinv + b[3:4, 0:16]
    ohi[...] = (agg[1] + t2h[...]) * dinv + b[3:4, 16:32]


def _row_spec(cols):
    return pl.BlockSpec((_BLK, cols), lambda i: (i, 0))


def _full_spec(shape):
    nd = len(shape)
    return pl.BlockSpec(shape, lambda i, _n=nd: (0,) * _n)


def _half_spec():
    return pl.BlockSpec((2, _BLK, 16), lambda i: (0, i, 0))


def _deg_spec():
    return pl.BlockSpec((2, _BLK, 1), lambda i: (0, i, 0))


_half_out = jax.ShapeDtypeStruct((_NUP, 16), jnp.float32)


def _densea(feat, img, wf, wi, w1, ball):
    return pl.pallas_call(
        _densea_body,
        grid=(_GRID,),
        in_specs=[_row_spec(32), _row_spec(32), _full_spec((32, 24)),
                  _full_spec((32, 8)), _full_spec((32, 32)),
                  _full_spec((8, 32))],
        out_specs=_half_spec(),
        out_shape=jax.ShapeDtypeStruct((2, _NUP, 16), jnp.float32),
    )(feat, img, wf, wi, w1, ball)


def _denseb(y1, degr):
    return pl.pallas_call(
        _denseb_body,
        grid=(_GRID,),
        in_specs=[_half_spec(), _deg_spec()],
        out_specs=[_row_spec(16), _row_spec(16)],
        out_shape=[_half_out, _half_out],
    )(y1, degr)


def _dense2(agg, t1l, t1h, degr, w2, ball):
    return pl.pallas_call(
        _dense2_body,
        grid=(_GRID,),
        in_specs=[_half_spec(), _row_spec(16), _row_spec(16), _deg_spec(),
                  _full_spec((32, 32)), _full_spec((8, 32))],
        out_specs=[_row_spec(16), _row_spec(16)],
        out_shape=[_half_out, _half_out],
    )(agg, t1l, t1h, degr, w2, ball)


def _dense3(agg, t2l, t2h, degr, ball):
    return pl.pallas_call(
        _dense3_body,
        grid=(_GRID,),
        in_specs=[_half_spec(), _row_spec(16), _row_spec(16), _deg_spec(),
                  _full_spec((8, 32))],
        out_specs=[_row_spec(16), _row_spec(16)],
        out_shape=[_half_out, _half_out],
    )(agg, t2l, t2h, degr, ball)


def kernel(feature, img, edge_index, W_feat, b_feat, W_img, b_img, W_g1, b_g1,
           W_g2, b_g2):
    src2d = jnp.concatenate(
        [edge_index[0].reshape(_EROWS, _L),
         jnp.zeros((_PADR, _L), jnp.int32)])
    dst2d = jnp.concatenate(
        [edge_index[1].reshape(_EROWS, _L),
         jnp.full((_PADR, _L), _N, jnp.int32)])

    ones_h = jnp.ones((_L,), jnp.float32)
    z1d = jnp.zeros((_ZCH,), jnp.float32)
    zrows = jnp.zeros((_ZCH, 16), jnp.float32)

    ball = jnp.stack([
        jnp.pad(b_feat, (0, 8)),
        jnp.pad(b_img, (0, 24)),
        b_g1,
        jnp.pad(b_g2, (0, 12)),
    ] + [jnp.zeros((32,), jnp.float32)] * 4)
    w2 = jnp.pad(W_g2, ((0, 0), (0, 12)))

    deg2 = _deg(dst2d, ones_h, z1d)
    degr = deg2.reshape(2, _NUP, 1)

    y1 = _densea(feature, img, W_feat, W_img, W_g1, ball)
    t1l, t1h = _denseb(y1, degr)
    agg1 = _scatter(src2d, dst2d, t1l, t1h, zrows)
    t2l, t2h = _dense2(agg1, t1l, t1h, degr, w2, ball)
    agg2 = _scatter(src2d, dst2d, t2l, t2h, zrows)
    olo, ohi = _dense3(agg2, t2l, t2h, degr, ball)
    return jnp.concatenate([olo[:_N], ohi[:_N, :4]], axis=1)


# speedup vs baseline: 35.4279x; 1.1963x over previous
import functools

import jax
import jax.numpy as jnp
from jax import lax
from jax.experimental import pallas as pl
from jax.experimental.pallas import tpu as pltpu
from jax.experimental.pallas import tpu_sc as plsc

_N = 100000
_E = 1600000

_L = 128
_EROWS = _E // _L
_PADR = 44
_RPT = 784
_SLAB = 8
_SLABS = _RPT // _SLAB
_H = _SLAB // 2

_DRPW = 392
_DCH = 49

_ZCH = 6272
_NUP = _ZCH * 16
_FTOT = _NUP * 16 // _L

_BLK = 2048
_FBLK = _BLK * 16 // _L
_GRID = _NUP // _BLK

_mesh = plsc.VectorSubcoreMesh(core_axis_name="c", subcore_axis_name="s",
                               num_cores=2, num_subcores=16)


def _writeback(acc, out, s):
    pltpu.sync_copy(acc.at[pl.ds(s * _ZCH, _ZCH)],
                    out.at[pl.ds(s * _ZCH, _ZCH)])


def _deg_body(dst2d, ones_h, z1d, out, didxA, didxB, ones_v, acc, isem, ssem):
    c = lax.axis_index("c")
    s = lax.axis_index("s")
    w = c * 16 + s
    pltpu.sync_copy(z1d, acc.at[pl.ds(s * _ZCH, _ZCH)])
    pltpu.sync_copy(ones_h, ones_v)
    plsc.subcore_barrier()
    base = w * _DRPW

    def idx_fetch(i, didx):
        pltpu.async_copy(dst2d.at[pl.ds(base + i * _SLAB, _SLAB)], didx, isem)

    def idx_wait(didx):
        pltpu.make_async_copy(dst2d.at[pl.ds(0, _SLAB)], didx, isem).wait()

    def drain_adds():
        for _ in range(_SLAB):
            pltpu.make_async_copy(ones_v, acc.at[pl.ds(0, _L)], ssem).wait()

    def chunk(i, didx, didxn, first):
        idx_wait(didx)
        if first is None:
            drain_adds()
        idx_fetch(lax.min(i + 1, _DCH - 1), didxn)
        for j in range(_SLAB):
            pltpu.async_copy(ones_v, acc.at[didx.at[j]], ssem, add=True)

    def body(i, carry):
        r0 = base + i * _SLAB
        pltpu.sync_copy(dst2d.at[pl.ds(r0, _SLAB)], didxA)
        for j in range(_SLAB):
            pltpu.sync_copy(ones_v, acc.at[didxA.at[j]], add=True)
        return carry

    lax.fori_loop(0, _DCH, body, 0)
    plsc.subcore_barrier()
    _writeback(acc, out.at[c], s)


@functools.partial(
    pl.kernel,
    out_type=jax.ShapeDtypeStruct((2, _NUP), jnp.float32),
    mesh=_mesh,
    scratch_types=[
        pltpu.VMEM((_SLAB, _L), jnp.int32),
        pltpu.VMEM((_SLAB, _L), jnp.int32),
        pltpu.VMEM((_L,), jnp.float32),
        pltpu.VMEM_SHARED((_NUP,), jnp.float32),
        pltpu.SemaphoreType.DMA,
        pltpu.SemaphoreType.DMA,
    ],
)
def _deg(dst2d, ones_h, z1d, out, didxA, didxB, ones_v, acc, isem, ssem):
    _deg_body(dst2d, ones_h, z1d, out, didxA, didxB, ones_v, acc, isem, ssem)


def _scatter_body(src2d, dst2d, tlo, thi, zrows, out, sidxA, didxA, sidxB,
                  didxB, rows0, rows1, acc, gsem0, gsem1, ssem0, ssem1, isem):
    c = lax.axis_index("c")
    s = lax.axis_index("s")
    pltpu.sync_copy(zrows, acc.at[pl.ds(s * _ZCH, _ZCH)])
    plsc.subcore_barrier()
    base = s * _RPT

    def idx_fetch(i, sidx, didx):
        r0 = base + i * _SLAB
        pltpu.async_copy(src2d.at[pl.ds(r0, _SLAB)], sidx, isem)
        pltpu.async_copy(dst2d.at[pl.ds(r0, _SLAB)], didx, isem)

    def idx_wait(sidx, didx):
        pltpu.make_async_copy(src2d.at[pl.ds(0, _SLAB)], sidx, isem).wait()
        pltpu.make_async_copy(dst2d.at[pl.ds(0, _SLAB)], didx, isem).wait()

    def run(table):
        def drain_g(rows, gsem):
            pltpu.make_async_copy(table.at[pl.ds(0, _H * _L)], rows,
                                  gsem).wait()

        def drain_s(rows, ssem):
            pltpu.make_async_copy(rows, acc.at[pl.ds(0, _H * _L)],
                                  ssem).wait()

        def do_slab(i, sidx, didx, sidxn, didxn, first):
            idx_wait(sidx, didx)

            def gathers(h, rows, gsem):
                for j in range(_H):
                    pltpu.async_copy(table.at[sidx.at[h * _H + j]],
                                     rows.at[pl.ds(j * _L, _L)], gsem)

            def scatters(h, rows, ssem):
                for j in range(_H):
                    pltpu.async_copy(rows.at[pl.ds(j * _L, _L)],
                                     acc.at[didx.at[h * _H + j]], ssem,
                                     add=True)

            if first is None:
                drain_s(rows0, ssem0)
            else:
                @pl.when(first)
                def _():
                    drain_s(rows0, ssem0)
            gathers(0, rows0, gsem0)
            if first is None:
                drain_s(rows1, ssem1)
            else:
                @pl.when(first)
                def _():
                    drain_s(rows1, ssem1)
            gathers(1, rows1, gsem1)
            idx_fetch(lax.min(i + 1, _SLABS - 1), sidxn, didxn)
            drain_g(rows0, gsem0)
            scatters(0, rows0, ssem0)
            drain_g(rows1, gsem1)
            scatters(1, rows1, ssem1)

        def body(k, carry):
            do_slab(2 * k, sidxA, didxA, sidxB, didxB, k > 0)
            do_slab(2 * k + 1, sidxB, didxB, sidxA, didxA, None)
            return carry

        idx_fetch(0, sidxA, didxA)
        lax.fori_loop(0, _SLABS // 2, body, 0)
        idx_wait(sidxA, didxA)
        drain_s(rows0, ssem0)
        drain_s(rows1, ssem1)

    @pl.when(c == 0)
    def _():
        run(tlo)

    @pl.when(c == 1)
    def _():
        run(thi)

    plsc.subcore_barrier()
    _writeback(acc, out.at[c], s)


@functools.partial(
    pl.kernel,
    out_type=jax.ShapeDtypeStruct((2, _NUP, 16), jnp.float32),
    mesh=_mesh,
    compiler_params=pltpu.CompilerParams(use_tc_tiling_on_sc=False),
    scratch_types=[
        pltpu.VMEM((_SLAB, _L), jnp.int32),
        pltpu.VMEM((_SLAB, _L), jnp.int32),
        pltpu.VMEM((_SLAB, _L), jnp.int32),
        pltpu.VMEM((_SLAB, _L), jnp.int32),
        pltpu.VMEM((_H * _L, 16), jnp.float32),
        pltpu.VMEM((_H * _L, 16), jnp.float32),
        pltpu.VMEM_SHARED((_NUP, 16), jnp.float32),
        pltpu.SemaphoreType.DMA,
        pltpu.SemaphoreType.DMA,
        pltpu.SemaphoreType.DMA,
        pltpu.SemaphoreType.DMA,
        pltpu.SemaphoreType.DMA,
    ],
)
def _scatter(src2d, dst2d, tlo, thi, zrows, out, sidxA, didxA, sidxB, didxB,
             rows0, rows1, acc, gsem0, gsem1, ssem0, ssem1, isem):
    _scatter_body(src2d, dst2d, tlo, thi, zrows, out, sidxA, didxA, sidxB,
                  didxB, rows0, rows1, acc, gsem0, gsem1, ssem0, ssem1, isem)


def _dot(a, b):
    return jnp.dot(a, b, preferred_element_type=jnp.float32,
                   precision=lax.Precision.HIGHEST)


def _dinv_of(degr):
    return lax.rsqrt(degr[0] + degr[1] + 1.0)


def _dense1_body(feat, img, degr, wf, wi, w1, ball, tlo, thi):
    dinv = _dinv_of(degr)
    b = ball[...]
    f = jnp.maximum(_dot(feat[...], wf[...]) + b[0:1, 0:24], 0.0)
    im = jnp.maximum(_dot(img[...], wi[...]) + b[1:2, 0:8], 0.0)
    w1v = w1[...]
    tlo[...] = (_dot(f, w1v[0:24, 0:16]) + _dot(im, w1v[24:32, 0:16])) * dinv
    thi[...] = (_dot(f, w1v[0:24, 16:32]) + _dot(im, w1v[24:32, 16:32])) * dinv


def _dinv8_of(deg8, b8):
    d = lax.rsqrt(deg8[0] + deg8[1] + 1.0)
    return _dot(d, b8)


def _dense2_body(aggf, t1l, t1h, deg8, b8, wll, wlh, whl, whh, bias, tlo, thi):
    dinv = _dinv8_of(deg8, b8[...])
    bv = bias[...]
    hl = jnp.maximum((aggf[0] + t1l[...]) * dinv + bv[0:1], 0.0)
    hh = jnp.maximum((aggf[1] + t1h[...]) * dinv + bv[1:2], 0.0)
    tlo[...] = (_dot(hl, wll[...]) + _dot(hh, whl[...])) * dinv
    thi[...] = (_dot(hl, wlh[...]) + _dot(hh, whh[...])) * dinv


def _dense3_body(aggf, t2l, t2h, deg8, b8, bias, olo, ohi):
    dinv = _dinv8_of(deg8, b8[...])
    bv = bias[...]
    olo[...] = (aggf[0] + t2l[...]) * dinv + bv[2:3]
    ohi[...] = (aggf[1] + t2h[...]) * dinv + bv[3:4]


def _row_spec(cols):
    return pl.BlockSpec((_BLK, cols), lambda i: (i, 0))


def _flat_spec():
    return pl.BlockSpec((_FBLK, _L), lambda i: (i, 0))


def _flat2_spec():
    return pl.BlockSpec((2, _FBLK, _L), lambda i: (0, i, 0))


def _full_spec(shape):
    nd = len(shape)
    return pl.BlockSpec(shape, lambda i, _n=nd: (0,) * _n)


def _deg_spec():
    return pl.BlockSpec((2, _BLK, 1), lambda i: (0, i, 0))


def _deg8_spec():
    return pl.BlockSpec((2, _FBLK, 8), lambda i: (0, i, 0))


_flat_out = jax.ShapeDtypeStruct((_FTOT, _L), jnp.float32)
_half_out = jax.ShapeDtypeStruct((_NUP, 16), jnp.float32)


def _dense1(feat, img, degr, wf, wi, w1, ball):
    return pl.pallas_call(
        _dense1_body,
        grid=(_GRID,),
        in_specs=[_row_spec(32), _row_spec(32), _deg_spec(),
                  _full_spec((32, 24)), _full_spec((32, 8)),
                  _full_spec((32, 32)), _full_spec((8, 32))],
        out_specs=[_row_spec(16), _row_spec(16)],
        out_shape=[_half_out, _half_out],
    )(feat, img, degr, wf, wi, w1, ball)


def _dense2(aggf, t1l, t1h, deg8, b8, wll, wlh, whl, whh, bias):
    return pl.pallas_call(
        _dense2_body,
        grid=(_GRID,),
        in_specs=[_flat2_spec(), _flat_spec(), _flat_spec(), _deg8_spec(),
                  _full_spec((8, _L)), _full_spec((_L, _L)),
                  _full_spec((_L, _L)), _full_spec((_L, _L)),
                  _full_spec((_L, _L)), _full_spec((8, _L))],
        out_specs=[_flat_spec(), _flat_spec()],
        out_shape=[_flat_out, _flat_out],
    )(aggf, t1l, t1h, deg8, b8, wll, wlh, whl, whh, bias)


def _dense3(aggf, t2l, t2h, deg8, b8, bias):
    return pl.pallas_call(
        _dense3_body,
        grid=(_GRID,),
        in_specs=[_flat2_spec(), _flat_spec(), _flat_spec(), _deg8_spec(),
                  _full_spec((8, _L)), _full_spec((8, _L))],
        out_specs=[_flat_spec(), _flat_spec()],
        out_shape=[_flat_out, _flat_out],
    )(aggf, t2l, t2h, deg8, b8, bias)


def kernel(feature, img, edge_index, W_feat, b_feat, W_img, b_img, W_g1, b_g1,
           W_g2, b_g2):
    e2d = edge_index.reshape(2, _EROWS, _L)
    epad = jnp.concatenate(
        [e2d, jnp.full((2, _PADR, _L), _N, jnp.int32)], axis=1)
    src2d = epad[0]
    dst2d = epad[1]

    ones_h = jnp.ones((_L,), jnp.float32)
    z1d = jnp.zeros((_ZCH,), jnp.float32)
    zrows = jnp.zeros((_ZCH, 16), jnp.float32)

    ball = jnp.stack([
        jnp.pad(b_feat, (0, 8)),
        jnp.pad(b_img, (0, 24)),
        b_g1,
        jnp.pad(b_g2, (0, 12)),
    ] + [jnp.zeros((32,), jnp.float32)] * 4)
    w2 = jnp.pad(W_g2, ((0, 0), (0, 12)))
    eye8 = jnp.eye(8, dtype=jnp.float32)
    wll = jnp.kron(eye8, w2[0:16, 0:16])
    wlh = jnp.kron(eye8, w2[0:16, 16:32])
    whl = jnp.kron(eye8, w2[16:32, 0:16])
    whh = jnp.kron(eye8, w2[16:32, 16:32])
    b8 = jnp.kron(eye8, jnp.ones((1, 16), jnp.float32))
    bias8 = jnp.stack([
        jnp.tile(b_g1[0:16], 8),
        jnp.tile(b_g1[16:32], 8),
        jnp.tile(b_g2[0:16], 8),
        jnp.tile(jnp.pad(b_g2[16:20], (0, 12)), 8),
    ] + [jnp.zeros((_L,), jnp.float32)] * 4)

    deg2 = _deg(dst2d, ones_h, z1d)
    degr = deg2.reshape(2, _NUP, 1)
    deg8 = deg2.reshape(2, _FTOT, 8)

    t1l, t1h = _dense1(feature, img, degr, W_feat, W_img, W_g1, ball)
    agg1 = _scatter(src2d, dst2d, t1l, t1h, zrows)
    t2l, t2h = _dense2(agg1.reshape(2, _FTOT, _L),
                       t1l.reshape(_FTOT, _L), t1h.reshape(_FTOT, _L),
                       deg8, b8, wll, wlh, whl, whh, bias8)
    agg2 = _scatter(src2d, dst2d, t2l.reshape(_NUP, 16),
                    t2h.reshape(_NUP, 16), zrows)
    olo, ohi = _dense3(agg2.reshape(2, _FTOT, _L), t2l, t2h, deg8, b8, bias8)
    return jnp.concatenate([olo.reshape(_NUP, 16)[:_N],
                            ohi.reshape(_NUP, 16)[:_N, 0:4]], axis=1)


# speedup vs baseline: 35.9949x; 1.0160x over previous
import functools

import jax
import jax.numpy as jnp
from jax import lax
from jax.experimental import pallas as pl
from jax.experimental.pallas import tpu as pltpu
from jax.experimental.pallas import tpu_sc as plsc

_N = 100000
_E = 1600000

_L = 128
_EROWS = _E // _L
_PADR = 44
_RPT = 784
_SLAB = 8
_SLABS = _RPT // _SLAB
_H = _SLAB // 2

_DRPW = 392
_DCH = 49

_ZCH = 6272
_NUP = _ZCH * 16
_FTOT = _NUP * 16 // _L

_BLK = 2048
_FBLK = _BLK * 16 // _L
_GRID = _NUP // _BLK

_mesh = plsc.VectorSubcoreMesh(core_axis_name="c", subcore_axis_name="s",
                               num_cores=2, num_subcores=16)


def _writeback(acc, out, s):
    pltpu.sync_copy(acc.at[pl.ds(s * _ZCH, _ZCH)],
                    out.at[pl.ds(s * _ZCH, _ZCH)])


def _deg_body(dst2d, ones_h, z1d, out, didxA, didxB, ones_v, acc, isem, ssem):
    c = lax.axis_index("c")
    s = lax.axis_index("s")
    w = c * 16 + s
    pltpu.sync_copy(z1d, acc.at[pl.ds(s * _ZCH, _ZCH)])
    pltpu.sync_copy(ones_h, ones_v)
    plsc.subcore_barrier()
    base = w * _DRPW

    def idx_fetch(i, didx):
        pltpu.async_copy(dst2d.at[pl.ds(base + i * _SLAB, _SLAB)], didx, isem)

    def idx_wait(didx):
        pltpu.make_async_copy(dst2d.at[pl.ds(0, _SLAB)], didx, isem).wait()

    def drain_adds():
        for _ in range(_SLAB):
            pltpu.make_async_copy(ones_v, acc.at[pl.ds(0, _L)], ssem).wait()

    def chunk(i, didx, didxn, first):
        idx_wait(didx)
        if first is None:
            drain_adds()
        idx_fetch(lax.min(i + 1, _DCH - 1), didxn)
        for j in range(_SLAB):
            pltpu.async_copy(ones_v, acc.at[didx.at[j]], ssem, add=True)

    def body(k, carry):
        chunk(2 * k + 1, didxB, didxA, None)
        chunk(2 * k + 2, didxA, didxB, None)
        return carry

    idx_fetch(0, didxA)
    chunk(0, didxA, didxB, False)
    lax.fori_loop(0, (_DCH - 1) // 2, body, 0)
    idx_wait(didxA)
    drain_adds()
    plsc.subcore_barrier()
    _writeback(acc, out.at[c], s)


@functools.partial(
    pl.kernel,
    out_type=jax.ShapeDtypeStruct((2, _NUP), jnp.float32),
    mesh=_mesh,
    scratch_types=[
        pltpu.VMEM((_SLAB, _L), jnp.int32),
        pltpu.VMEM((_SLAB, _L), jnp.int32),
        pltpu.VMEM((_L,), jnp.float32),
        pltpu.VMEM_SHARED((_NUP,), jnp.float32),
        pltpu.SemaphoreType.DMA,
        pltpu.SemaphoreType.DMA,
    ],
)
def _deg(dst2d, ones_h, z1d, out, didxA, didxB, ones_v, acc, isem, ssem):
    _deg_body(dst2d, ones_h, z1d, out, didxA, didxB, ones_v, acc, isem, ssem)


def _scatter_body(src2d, dst2d, tlo, thi, zrows, out, sidxA, didxA, sidxB,
                  didxB, rows0, rows1, acc, gsem0, gsem1, ssem0, ssem1, isem):
    c = lax.axis_index("c")
    s = lax.axis_index("s")
    pltpu.sync_copy(zrows, acc.at[pl.ds(s * _ZCH, _ZCH)])
    plsc.subcore_barrier()
    base = s * _RPT

    def idx_fetch(i, sidx, didx):
        r0 = base + i * _SLAB
        pltpu.async_copy(src2d.at[pl.ds(r0, _SLAB)], sidx, isem)
        pltpu.async_copy(dst2d.at[pl.ds(r0, _SLAB)], didx, isem)

    def idx_wait(sidx, didx):
        pltpu.make_async_copy(src2d.at[pl.ds(0, _SLAB)], sidx, isem).wait()
        pltpu.make_async_copy(dst2d.at[pl.ds(0, _SLAB)], didx, isem).wait()

    def run(table):
        def drain_g(rows, gsem):
            pltpu.make_async_copy(table.at[pl.ds(0, _H * _L)], rows,
                                  gsem).wait()

        def drain_s(rows, ssem):
            pltpu.make_async_copy(rows, acc.at[pl.ds(0, _H * _L)],
                                  ssem).wait()

        def do_slab(i, sidx, didx, sidxn, didxn, first):
            idx_wait(sidx, didx)

            def gathers(h, rows, gsem):
                for j in range(_H):
                    pltpu.async_copy(table.at[sidx.at[h * _H + j]],
                                     rows.at[pl.ds(j * _L, _L)], gsem)

            def scatters(h, rows, ssem):
                for j in range(_H):
                    pltpu.async_copy(rows.at[pl.ds(j * _L, _L)],
                                     acc.at[didx.at[h * _H + j]], ssem,
                                     add=True)

            if first is None:
                drain_s(rows0, ssem0)
            else:
                @pl.when(first)
                def _():
                    drain_s(rows0, ssem0)
            gathers(0, rows0, gsem0)
            if first is None:
                drain_s(rows1, ssem1)
            else:
                @pl.when(first)
                def _():
                    drain_s(rows1, ssem1)
            gathers(1, rows1, gsem1)
            idx_fetch(lax.min(i + 1, _SLABS - 1), sidxn, didxn)
            drain_g(rows0, gsem0)
            scatters(0, rows0, ssem0)
            drain_g(rows1, gsem1)
            scatters(1, rows1, ssem1)

        def body(k, carry):
            do_slab(2 * k, sidxA, didxA, sidxB, didxB, k > 0)
            do_slab(2 * k + 1, sidxB, didxB, sidxA, didxA, None)
            return carry

        idx_fetch(0, sidxA, didxA)
        lax.fori_loop(0, _SLABS // 2, body, 0)
        idx_wait(sidxA, didxA)
        drain_s(rows0, ssem0)
        drain_s(rows1, ssem1)

    @pl.when(c == 0)
    def _():
        run(tlo)

    @pl.when(c == 1)
    def _():
        run(thi)

    plsc.subcore_barrier()
    _writeback(acc, out.at[c], s)


@functools.partial(
    pl.kernel,
    out_type=jax.ShapeDtypeStruct((2, _NUP, 16), jnp.float32),
    mesh=_mesh,
    compiler_params=pltpu.CompilerParams(use_tc_tiling_on_sc=False),
    scratch_types=[
        pltpu.VMEM((_SLAB, _L), jnp.int32),
        pltpu.VMEM((_SLAB, _L), jnp.int32),
        pltpu.VMEM((_SLAB, _L), jnp.int32),
        pltpu.VMEM((_SLAB, _L), jnp.int32),
        pltpu.VMEM((_H * _L, 16), jnp.float32),
        pltpu.VMEM((_H * _L, 16), jnp.float32),
        pltpu.VMEM_SHARED((_NUP, 16), jnp.float32),
        pltpu.SemaphoreType.DMA,
        pltpu.SemaphoreType.DMA,
        pltpu.SemaphoreType.DMA,
        pltpu.SemaphoreType.DMA,
        pltpu.SemaphoreType.DMA,
    ],
)
def _scatter(src2d, dst2d, tlo, thi, zrows, out, sidxA, didxA, sidxB, didxB,
             rows0, rows1, acc, gsem0, gsem1, ssem0, ssem1, isem):
    _scatter_body(src2d, dst2d, tlo, thi, zrows, out, sidxA, didxA, sidxB,
                  didxB, rows0, rows1, acc, gsem0, gsem1, ssem0, ssem1, isem)


def _dot(a, b):
    return jnp.dot(a, b, preferred_element_type=jnp.float32,
                   precision=lax.Precision.HIGHEST)


def _dinv_of(degr):
    return lax.rsqrt(degr[0] + degr[1] + 1.0)


def _dense1_body(feat, img, degr, wf, wi, w1, ball, tlo, thi):
    dinv = _dinv_of(degr)
    b = ball[...]
    f = jnp.maximum(_dot(feat[...], wf[...]) + b[0:1, 0:24], 0.0)
    im = jnp.maximum(_dot(img[...], wi[...]) + b[1:2, 0:8], 0.0)
    w1v = w1[...]
    tlo[...] = (_dot(f, w1v[0:24, 0:16]) + _dot(im, w1v[24:32, 0:16])) * dinv
    thi[...] = (_dot(f, w1v[0:24, 16:32]) + _dot(im, w1v[24:32, 16:32])) * dinv


def _dinv8_of(deg8, b8):
    d = lax.rsqrt(deg8[0] + deg8[1] + 1.0)
    return _dot(d, b8)


def _dense2_body(aggf, t1l, t1h, deg8, b8, wll, wlh, whl, whh, bias, tlo, thi):
    dinv = _dinv8_of(deg8, b8[...])
    bv = bias[...]
    hl = jnp.maximum((aggf[0] + t1l[...]) * dinv + bv[0:1], 0.0)
    hh = jnp.maximum((aggf[1] + t1h[...]) * dinv + bv[1:2], 0.0)
    tlo[...] = (_dot(hl, wll[...]) + _dot(hh, whl[...])) * dinv
    thi[...] = (_dot(hl, wlh[...]) + _dot(hh, whh[...])) * dinv


def _dense3_body(aggf, t2l, t2h, deg8, b8, bias, olo, ohi):
    dinv = _dinv8_of(deg8, b8[...])
    bv = bias[...]
    olo[...] = (aggf[0] + t2l[...]) * dinv + bv[2:3]
    ohi[...] = (aggf[1] + t2h[...]) * dinv + bv[3:4]


def _row_spec(cols):
    return pl.BlockSpec((_BLK, cols), lambda i: (i, 0))


def _flat_spec():
    return pl.BlockSpec((_FBLK, _L), lambda i: (i, 0))


def _flat2_spec():
    return pl.BlockSpec((2, _FBLK, _L), lambda i: (0, i, 0))


def _full_spec(shape):
    nd = len(shape)
    return pl.BlockSpec(shape, lambda i, _n=nd: (0,) * _n)


def _deg_spec():
    return pl.BlockSpec((2, _BLK, 1), lambda i: (0, i, 0))


def _deg8_spec():
    return pl.BlockSpec((2, _FBLK, 8), lambda i: (0, i, 0))


_flat_out = jax.ShapeDtypeStruct((_FTOT, _L), jnp.float32)
_half_out = jax.ShapeDtypeStruct((_NUP, 16), jnp.float32)


def _dense1(feat, img, degr, wf, wi, w1, ball):
    return pl.pallas_call(
        _dense1_body,
        grid=(_GRID,),
        in_specs=[_row_spec(32), _row_spec(32), _deg_spec(),
                  _full_spec((32, 24)), _full_spec((32, 8)),
                  _full_spec((32, 32)), _full_spec((8, 32))],
        out_specs=[_row_spec(16), _row_spec(16)],
        out_shape=[_half_out, _half_out],
    )(feat, img, degr, wf, wi, w1, ball)


def _dense2(aggf, t1l, t1h, deg8, b8, wll, wlh, whl, whh, bias):
    return pl.pallas_call(
        _dense2_body,
        grid=(_GRID,),
        in_specs=[_flat2_spec(), _flat_spec(), _flat_spec(), _deg8_spec(),
                  _full_spec((8, _L)), _full_spec((_L, _L)),
                  _full_spec((_L, _L)), _full_spec((_L, _L)),
                  _full_spec((_L, _L)), _full_spec((8, _L))],
        out_specs=[_flat_spec(), _flat_spec()],
        out_shape=[_flat_out, _flat_out],
    )(aggf, t1l, t1h, deg8, b8, wll, wlh, whl, whh, bias)


def _dense3(aggf, t2l, t2h, deg8, b8, bias):
    return pl.pallas_call(
        _dense3_body,
        grid=(_GRID,),
        in_specs=[_flat2_spec(), _flat_spec(), _flat_spec(), _deg8_spec(),
                  _full_spec((8, _L)), _full_spec((8, _L))],
        out_specs=[_flat_spec(), _flat_spec()],
        out_shape=[_flat_out, _flat_out],
    )(aggf, t2l, t2h, deg8, b8, bias)


def kernel(feature, img, edge_index, W_feat, b_feat, W_img, b_img, W_g1, b_g1,
           W_g2, b_g2):
    e2d = edge_index.reshape(2, _EROWS, _L)
    epad = jnp.concatenate(
        [e2d, jnp.full((2, _PADR, _L), _N, jnp.int32)], axis=1)
    src2d = epad[0]
    dst2d = epad[1]

    ones_h = jnp.ones((_L,), jnp.float32)
    z1d = jnp.zeros((_ZCH,), jnp.float32)
    zrows = jnp.zeros((_ZCH, 16), jnp.float32)

    ball = jnp.stack([
        jnp.pad(b_feat, (0, 8)),
        jnp.pad(b_img, (0, 24)),
        b_g1,
        jnp.pad(b_g2, (0, 12)),
    ] + [jnp.zeros((32,), jnp.float32)] * 4)
    w2 = jnp.pad(W_g2, ((0, 0), (0, 12)))
    eye8 = jnp.eye(8, dtype=jnp.float32)
    wll = jnp.kron(eye8, w2[0:16, 0:16])
    wlh = jnp.kron(eye8, w2[0:16, 16:32])
    whl = jnp.kron(eye8, w2[16:32, 0:16])
    whh = jnp.kron(eye8, w2[16:32, 16:32])
    b8 = jnp.kron(eye8, jnp.ones((1, 16), jnp.float32))
    bias8 = jnp.stack([
        jnp.tile(b_g1[0:16], 8),
        jnp.tile(b_g1[16:32], 8),
        jnp.tile(b_g2[0:16], 8),
        jnp.tile(jnp.pad(b_g2[16:20], (0, 12)), 8),
    ] + [jnp.zeros((_L,), jnp.float32)] * 4)

    deg2 = _deg(dst2d, ones_h, z1d)
    degr = deg2.reshape(2, _NUP, 1)
    deg8 = deg2.reshape(2, _FTOT, 8)

    t1l, t1h = _dense1(feature, img, degr, W_feat, W_img, W_g1, ball)
    agg1 = _scatter(src2d, dst2d, t1l, t1h, zrows)
    t2l, t2h = _dense2(agg1.reshape(2, _FTOT, _L),
                       t1l.reshape(_FTOT, _L), t1h.reshape(_FTOT, _L),
                       deg8, b8, wll, wlh, whl, whh, bias8)
    agg2 = _scatter(src2d, dst2d, t2l.reshape(_NUP, 16),
                    t2h.reshape(_NUP, 16), zrows)
    olo, ohi = _dense3(agg2.reshape(2, _FTOT, _L), t2l, t2h, deg8, b8, bias8)
    return jnp.concatenate([olo.reshape(_NUP, 16)[:_N],
                            ohi.reshape(_NUP, 16)[:_N, 0:4]], axis=1)
